# Initial kernel scaffold; baseline (speedup 1.0000x reference)
import jax
import jax.numpy as jnp
from jax.experimental import pallas as pl


def kernel(inputs, w, b):
    raise NotImplementedError("write your pallas kernel here")


# speedup vs baseline: 1.9251x; 1.9251x over previous
import functools

import jax
import jax.numpy as jnp
from jax import lax
from jax.experimental import pallas as pl
from jax.experimental.pallas import tpu as pltpu
from jax.experimental.pallas import tpu_sc as plsc

_INPUT_DIM = 1000000
_NUM_FIELDS = 26
_BATCH = 16384
_NC = 2
_NS = 16
_NW = _NC * _NS
_BPW = _BATCH // _NW
_CHUNK = _BPW * _NUM_FIELDS
_LANES = 16


def _sc_body(idx_hbm, w_hbm, bias_hbm, out_hbm, idx_v, vals_v, out_v, b_v, sem):
    cid = lax.axis_index("c")
    sid = lax.axis_index("s")
    wid = sid * _NC + cid

    pltpu.sync_copy(idx_hbm.at[wid], idx_v)
    pltpu.sync_copy(bias_hbm, b_v)
    pltpu.async_copy(w_hbm.at[idx_v], vals_v, sem).wait()

    bvec = b_v[...]

    def jbody(j, carry):
        base = j * _LANES
        acc = bvec
        for f in range(_NUM_FIELDS):
            acc = acc + vals_v[pl.ds(f * _BPW + base, _LANES)]
        out_v[pl.ds(base, _LANES)] = acc
        return carry

    lax.fori_loop(0, _BPW // _LANES, jbody, 0)
    pltpu.sync_copy(out_v, out_hbm.at[pl.ds(wid * _BPW, _BPW)])


@jax.jit
def _lr_logits(idx, w, bvec):
    mesh = plsc.VectorSubcoreMesh(core_axis_name="c", subcore_axis_name="s")
    ker = pl.kernel(
        _sc_body,
        out_type=jax.ShapeDtypeStruct((_BATCH,), jnp.float32),
        mesh=mesh,
        scratch_types=[
            pltpu.VMEM((_CHUNK,), jnp.int32),
            pltpu.VMEM((_CHUNK,), jnp.float32),
            pltpu.VMEM((_BPW,), jnp.float32),
            pltpu.VMEM((_LANES,), jnp.float32),
            pltpu.SemaphoreType.DMA,
        ],
    )
    return ker(idx, w, bvec)


def kernel(inputs, w, b):
    idx = inputs.reshape(_NW, _BPW, _NUM_FIELDS).transpose(0, 2, 1).reshape(_NW, _CHUNK)
    bvec = jnp.broadcast_to(b.astype(jnp.float32), (_LANES,))
    out = _lr_logits(idx, w, bvec)
    return out.reshape(_BATCH, 1)
